# Initial kernel scaffold; baseline (speedup 1.0000x reference)
import jax
import jax.numpy as jnp
from jax.experimental import pallas as pl


def kernel(x, a, v, params, eps):
    raise NotImplementedError("write your pallas kernel here")


# speedup vs baseline: 1.2927x; 1.2927x over previous
import functools

import jax
import jax.numpy as jnp
from jax.experimental import pallas as pl

B, N, D, H = 8, 1024, 32, 128
NUM_MP_STEPS = 2
INNER_ROUNDS = 5


def _gfvae_kernel(
    x_ref, a_ref, v_ref, eps_ref,
    wm1_0, bm1_0, wm2_0, bm2_0, wu1_0, bu1_0, wu2_0, bu2_0,
    wm1_1, bm1_1, wm2_1, bm2_1, wu1_1, bu1_1, wu2_1, bu2_1,
    w1, b1, w2, b2, w3m, w3s, b3m, b3s,
    ws, wt, bep,
    z_ref, nkl_ref, eplp_ref,
):
    f32 = jnp.float32
    xb = x_ref[0]
    ab = a_ref[0]
    nv = v_ref[0, 0, 0]

    mp = [
        (wm1_0, bm1_0, wm2_0, bm2_0, wu1_0, bu1_0, wu2_0, bu2_0),
        (wm1_1, bm1_1, wm2_1, bm2_1, wu1_1, bu1_1, wu2_1, bu2_1),
    ]

    dot = functools.partial(jnp.dot, preferred_element_type=f32)

    for (wm1, bm1, wm2, bm2, wu1, bu1, wu2, bu2) in mp:
        for _ in range(INNER_ROUNDS):
            h = jnp.tanh(dot(xb, wm1[...]) + bm1[0])
            m = jnp.tanh(dot(h, wm2[...]) + bm2[0])
            agg = dot(ab, m)
            u = jnp.concatenate([xb, agg], axis=-1)
            h2 = jnp.tanh(dot(u, wu1[...]) + bu1[0])
            xb = xb + jnp.tanh(dot(h2, wu2[...]) + bu2[0])

    he = jnp.tanh(dot(xb, w1[...]) + b1[0])
    he = jnp.tanh(dot(he, w2[...]) + b2[0])
    mean = dot(he, w3m[...]) + b3m[0]
    log_sd = dot(he, w3s[...]) + b3s[0]
    sd = jnp.exp(log_sd)

    rowmask = (
        jax.lax.broadcasted_iota(jnp.int32, (N, 1), 0).astype(f32) < nv
    ).astype(f32)

    kl = -log_sd + 0.5 * (sd * sd + mean * mean) - 0.5
    kl_sum = jnp.sum(kl * rowmask)
    neg_kl = -(kl_sum * (1.0 / (N * D)) * nv)

    z = mean + sd * eps_ref[0]
    z_ref[0] = z

    zs = dot(z, ws[...])
    zt = dot(z, wt[...])
    logits = jax.lax.dot_general(
        zs, zt, (((1,), (1,)), ((), ())), preferred_element_type=f32
    ) + bep[0, 0, 0]
    sp = jnp.maximum(logits, 0.0) + jnp.log1p(jnp.exp(-jnp.abs(logits)))
    lp = ab * logits - sp
    colmask = (
        jax.lax.broadcasted_iota(jnp.int32, (1, N), 1).astype(f32) < nv
    ).astype(f32)
    lp_sum = jnp.sum(lp * rowmask * colmask)
    eplp = lp_sum / (nv * nv)

    nkl_ref[0, 0, :] = jnp.broadcast_to(neg_kl, (128,))
    eplp_ref[0, 0, :] = jnp.broadcast_to(eplp, (128,))


def _full(shape):
    return pl.BlockSpec(shape, lambda b: (0,) * len(shape))


@jax.jit
def _run(x, a, v, params, eps):
    f32 = jnp.float32
    v3 = v.reshape(B, 1, 1).astype(f32)

    ops = [x, a, v3, eps]
    specs = [
        pl.BlockSpec((1, N, D), lambda b: (b, 0, 0)),
        pl.BlockSpec((1, N, N), lambda b: (b, 0, 0)),
        pl.BlockSpec((1, 1, 1), lambda b: (b, 0, 0)),
        pl.BlockSpec((1, N, D), lambda b: (b, 0, 0)),
    ]

    for p in params['mp']:
        step_ops = [
            p['Wm1'], p['bm1'].reshape(1, H), p['Wm2'], p['bm2'].reshape(1, D),
            p['Wu1'], p['bu1'].reshape(1, H), p['Wu2'],
            p['bu2'].reshape(1, D),
        ]
        ops += step_ops
        specs += [_full(o.shape) for o in step_ops]

    e = params['enc']
    enc_ops = [
        e['W1'], e['b1'].reshape(1, H), e['W2'], e['b2'].reshape(1, H),
        e['W3'][:, :D], e['W3'][:, D:], e['b3'][:D].reshape(1, D),
        e['b3'][D:].reshape(1, D),
    ]
    ops += enc_ops
    specs += [_full(o.shape) for o in enc_ops]

    ep = params['ep']
    ep_ops = [ep['Ws'], ep['Wt'], ep['b'].reshape(1, 1, 1)]
    ops += ep_ops
    specs += [_full(o.shape) for o in ep_ops]

    z, nkl, eplp = pl.pallas_call(
        _gfvae_kernel,
        grid=(B,),
        in_specs=specs,
        out_specs=[
            pl.BlockSpec((1, N, D), lambda b: (b, 0, 0)),
            pl.BlockSpec((1, 1, 128), lambda b: (b, 0, 0)),
            pl.BlockSpec((1, 1, 128), lambda b: (b, 0, 0)),
        ],
        out_shape=[
            jax.ShapeDtypeStruct((B, N, D), f32),
            jax.ShapeDtypeStruct((B, 1, 128), f32),
            jax.ShapeDtypeStruct((B, 1, 128), f32),
        ],
    )(*ops)

    return z, nkl[:, 0, 0], eplp[:, 0, 0]


def kernel(x, a, v, params, eps):
    return _run(x, a, v, params, eps)
